# Initial kernel scaffold; baseline (speedup 1.0000x reference)
import functools

import jax
import jax.numpy as jnp
from jax import lax
from jax.experimental import pallas as pl
from jax.experimental.pallas import tpu as pltpu
from jax.experimental.pallas import tpu_sc as plsc

NC = 2
NS = 16
NW = NC * NS

N_NODES = 10000
N_PAD = 10240
ROWS_PER_SUB = N_PAD // NS
E_EDGES = 320000
CHUNK = 128
CHUNKS_PER_TILE = 80
E_PAD = NW * CHUNKS_PER_TILE * CHUNK
D = 128


def _fill_zero_row(ref):
    for j in range(D // 16):
        ref[0, pl.ds(j * 16, 16)] = jnp.zeros((16,), jnp.float32)


def _agg_kernel_body(with_count, h_hbm, srcr, dstr, out_hbm, *rest):
    if with_count:
        cnt_out, src_v, dst_v, rows_v, zbuf, ones_v, zcnt, acc_sh, cnt_sh, sem = rest
    else:
        src_v, dst_v, rows_v, zbuf, acc_sh, sem = rest
    c = lax.axis_index("c")
    s = lax.axis_index("s")
    wid = s * NC + c

    _fill_zero_row(zbuf)
    k = 1
    while k < CHUNK:
        pltpu.sync_copy(zbuf.at[pl.ds(0, k)], zbuf.at[pl.ds(k, k)])
        k *= 2

    for k in range(ROWS_PER_SUB // CHUNK):
        pltpu.sync_copy(zbuf, acc_sh.at[pl.ds(s * ROWS_PER_SUB + k * CHUNK, CHUNK)])

    if with_count:
        for j in range(CHUNK // 16):
            ones_v[pl.ds(j * 16, 16)] = jnp.ones((16,), jnp.float32)

        def zc(i, _):
            zcnt[pl.ds(i * 16, 16)] = jnp.zeros((16,), jnp.float32)
            return 0
        lax.fori_loop(0, ROWS_PER_SUB // 16, zc, 0)
        pltpu.sync_copy(zcnt, cnt_sh.at[pl.ds(s * ROWS_PER_SUB, ROWS_PER_SUB)])

    plsc.subcore_barrier()

    pltpu.sync_copy(srcr.at[wid], src_v)
    pltpu.sync_copy(dstr.at[wid], dst_v)

    def chunk_step(j, _):
        pltpu.async_copy(h_hbm.at[src_v.at[j]], rows_v, sem).wait()
        pltpu.sync_copy(rows_v, acc_sh.at[dst_v.at[j]], add=True)
        if with_count:
            pltpu.sync_copy(ones_v, cnt_sh.at[dst_v.at[j]], add=True)
        return 0

    lax.fori_loop(0, CHUNKS_PER_TILE, chunk_step, 0)

    plsc.subcore_barrier()

    base = s * ROWS_PER_SUB
    pltpu.sync_copy(acc_sh.at[pl.ds(base, ROWS_PER_SUB)],
                    out_hbm.at[c, pl.ds(base, ROWS_PER_SUB)])
    if with_count:
        pltpu.sync_copy(cnt_sh.at[pl.ds(base, ROWS_PER_SUB)],
                        cnt_out.at[c, pl.ds(base, ROWS_PER_SUB)])


@functools.lru_cache(maxsize=None)
def _make_agg(n_rows, with_count):
    del n_rows
    mesh = plsc.VectorSubcoreMesh(core_axis_name="c", subcore_axis_name="s")
    out_type = [jax.ShapeDtypeStruct((NC, N_PAD, D), jnp.float32)]
    if with_count:
        out_type.append(jax.ShapeDtypeStruct((NC, N_PAD), jnp.float32))
    scratch = [
        pltpu.VMEM((CHUNKS_PER_TILE, CHUNK), jnp.int32),
        pltpu.VMEM((CHUNKS_PER_TILE, CHUNK), jnp.int32),
        pltpu.VMEM((CHUNK, D), jnp.float32),
        pltpu.VMEM((CHUNK, D), jnp.float32),
    ]
    if with_count:
        scratch.append(pltpu.VMEM((CHUNK,), jnp.float32))
        scratch.append(pltpu.VMEM((ROWS_PER_SUB,), jnp.float32))
    scratch.append(pltpu.VMEM_SHARED((N_PAD, D), jnp.float32))
    if with_count:
        scratch.append(pltpu.VMEM_SHARED((N_PAD,), jnp.float32))
    scratch.append(pltpu.SemaphoreType.DMA)

    return pl.kernel(
        functools.partial(_agg_kernel_body, with_count),
        mesh=mesh,
        out_type=tuple(out_type) if with_count else out_type[0],
        scratch_types=scratch,
    )


def _tc_layer_body(relu, acc_ref, cnt_ref, x_ref, wl_ref, wr_ref, b_ref, out_ref):
    a = acc_ref[0] + acc_ref[1]
    cnt = cnt_ref[0] + cnt_ref[1]
    rec = 1.0 / jnp.maximum(cnt, 1.0)
    mean = a * rec
    z = (jnp.dot(mean, wl_ref[...], preferred_element_type=jnp.float32)
         + jnp.dot(x_ref[...], wr_ref[...], preferred_element_type=jnp.float32)
         + b_ref[...])
    out_ref[...] = jnp.maximum(z, 0.0) if relu else z


@functools.lru_cache(maxsize=None)
def _make_tc_layer(relu):
    B = 1024
    grid = (N_PAD // B,)
    return pl.pallas_call(
        functools.partial(_tc_layer_body, relu),
        grid=grid,
        in_specs=[
            pl.BlockSpec((NC, B, D), lambda i: (0, i, 0)),
            pl.BlockSpec((NC, B, 1), lambda i: (0, i, 0)),
            pl.BlockSpec((B, D), lambda i: (i, 0)),
            pl.BlockSpec((D, D), lambda i: (0, 0)),
            pl.BlockSpec((D, D), lambda i: (0, 0)),
            pl.BlockSpec((1, D), lambda i: (0, 0)),
        ],
        out_specs=pl.BlockSpec((B, D), lambda i: (i, 0)),
        out_shape=jax.ShapeDtypeStruct((N_PAD, D), jnp.float32),
    )


def kernel(x, edge_index, W_l1, W_r1, b1, W_l2, W_r2, b2, W_l3, W_r3, b3):
    src = edge_index[0]
    dst = edge_index[1]
    pad_e = E_PAD - E_EDGES
    src_r = jnp.concatenate(
        [src, jnp.zeros((pad_e,), jnp.int32)]).reshape(NW, CHUNKS_PER_TILE, CHUNK)
    dst_r = jnp.concatenate(
        [dst, jnp.full((pad_e,), N_NODES, jnp.int32)]).reshape(NW, CHUNKS_PER_TILE, CHUNK)

    x_pad = jnp.pad(x, ((0, N_PAD - N_NODES), (0, 0)))

    a1, cnt = _make_agg(N_NODES, True)(x, src_r, dst_r)
    cnt3 = cnt.reshape(NC, N_PAD, 1)

    def pad_w(w):
        return jnp.pad(w, ((0, 0), (0, D - w.shape[1])))

    b1r = b1.reshape(1, D)
    b2r = b2.reshape(1, D)
    b3r = jnp.pad(b3, (0, D - b3.shape[0])).reshape(1, D)

    h1 = _make_tc_layer(True)(a1, cnt3, x_pad, W_l1, W_r1, b1r)
    a2 = _make_agg(N_PAD, False)(h1, src_r, dst_r)
    h2 = _make_tc_layer(True)(a2, cnt3, h1, W_l2, W_r2, b2r)
    a3 = _make_agg(N_PAD, False)(h2, src_r, dst_r)
    out = _make_tc_layer(False)(a3, cnt3, h2, pad_w(W_l3), pad_w(W_r3), b3r)
    return out[:N_NODES, :47]


# speedup vs baseline: 2.0480x; 2.0480x over previous
import functools

import jax
import jax.numpy as jnp
from jax import lax
from jax.experimental import pallas as pl
from jax.experimental.pallas import tpu as pltpu
from jax.experimental.pallas import tpu_sc as plsc

NC = 2
NS = 16
NW = NC * NS

N_NODES = 10000
N_PAD = 10240
HALF = N_PAD // NC
ROWS_PER_SUB = HALF // NS
ACC_ROWS = HALF + 256
E_EDGES = 320000
CHUNK = 128
CHUNKS_PER_TILE = 160
E_PAD = NS * CHUNKS_PER_TILE * CHUNK
D = 128


def _agg_kernel_body(with_count, h_hbm, srcr, dstr, *rest):
    if with_count:
        (out_hbm, cnt_out, src_v, dst_v, rows_v, zbuf,
         ones_v, zcnt, acc_sh, cnt_sh, sem) = rest
    else:
        out_hbm, src_v, dst_v, rows_v, zbuf, acc_sh, sem = rest
    c = lax.axis_index("c")
    s = lax.axis_index("s")
    base = s * ROWS_PER_SUB
    node_base = c * HALF

    def zb(i, _):
        for j in range(D // 16):
            zbuf[i, pl.ds(j * 16, 16)] = jnp.zeros((16,), jnp.float32)
        return 0
    lax.fori_loop(0, CHUNK, zb, 0)

    if with_count:
        for j in range(CHUNK // 16):
            ones_v[pl.ds(j * 16, 16)] = jnp.ones((16,), jnp.float32)

        def zc(i, _):
            zcnt[pl.ds(i * 16, 16)] = jnp.zeros((16,), jnp.float32)
            return 0
        lax.fori_loop(0, ROWS_PER_SUB // 16, zc, 0)
        pltpu.sync_copy(zcnt, cnt_sh.at[pl.ds(base, ROWS_PER_SUB)])

    pltpu.sync_copy(zbuf, acc_sh.at[pl.ds(base, CHUNK)])
    pltpu.sync_copy(zbuf, acc_sh.at[pl.ds(base + CHUNK, CHUNK)])
    pltpu.sync_copy(zbuf.at[pl.ds(0, ROWS_PER_SUB - 2 * CHUNK)],
                    acc_sh.at[pl.ds(base + 2 * CHUNK, ROWS_PER_SUB - 2 * CHUNK)])

    pltpu.sync_copy(srcr.at[s], src_v)
    pltpu.sync_copy(dstr.at[s], dst_v)

    def remap(i, _):
        for j in range(CHUNK // 16):
            d = dst_v[i, pl.ds(j * 16, 16)] - node_base
            ok = (d >= 0) & (d < HALF)
            dst_v[i, pl.ds(j * 16, 16)] = jnp.where(ok, d, HALF)
        return 0
    lax.fori_loop(0, CHUNKS_PER_TILE, remap, 0)

    plsc.subcore_barrier()

    def chunk_step(j, _):
        pltpu.async_copy(h_hbm.at[src_v.at[j]], rows_v, sem).wait()
        pltpu.sync_copy(rows_v, acc_sh.at[dst_v.at[j]], add=True)
        if with_count:
            pltpu.sync_copy(ones_v, cnt_sh.at[dst_v.at[j]], add=True)
        return 0

    lax.fori_loop(0, CHUNKS_PER_TILE, chunk_step, 0)

    plsc.subcore_barrier()

    for k in range(ROWS_PER_SUB // CHUNK):
        pltpu.sync_copy(acc_sh.at[pl.ds(base + k * CHUNK, CHUNK)], rows_v)
        pltpu.sync_copy(rows_v,
                        out_hbm.at[pl.ds(node_base + base + k * CHUNK, CHUNK)])
    rem = ROWS_PER_SUB % CHUNK
    if rem:
        done = (ROWS_PER_SUB // CHUNK) * CHUNK
        pltpu.sync_copy(acc_sh.at[pl.ds(base + done, rem)],
                        rows_v.at[pl.ds(0, rem)])
        pltpu.sync_copy(rows_v.at[pl.ds(0, rem)],
                        out_hbm.at[pl.ds(node_base + base + done, rem)])
    if with_count:
        pltpu.sync_copy(cnt_sh.at[pl.ds(base, ROWS_PER_SUB)], zcnt)
        pltpu.sync_copy(zcnt, cnt_out.at[pl.ds(node_base + base, ROWS_PER_SUB)])


@functools.lru_cache(maxsize=None)
def _make_agg(with_count):
    mesh = plsc.VectorSubcoreMesh(core_axis_name="c", subcore_axis_name="s")
    out_type = [jax.ShapeDtypeStruct((N_PAD, D), jnp.float32)]
    if with_count:
        out_type.append(jax.ShapeDtypeStruct((N_PAD,), jnp.float32))
    scratch = [
        pltpu.VMEM((CHUNKS_PER_TILE, CHUNK), jnp.int32),
        pltpu.VMEM((CHUNKS_PER_TILE, CHUNK), jnp.int32),
        pltpu.VMEM((CHUNK, D), jnp.float32),
        pltpu.VMEM((CHUNK, D), jnp.float32),
    ]
    if with_count:
        scratch.append(pltpu.VMEM((CHUNK,), jnp.float32))
        scratch.append(pltpu.VMEM((ROWS_PER_SUB,), jnp.float32))
    scratch.append(pltpu.VMEM_SHARED((ACC_ROWS, D), jnp.float32))
    if with_count:
        scratch.append(pltpu.VMEM_SHARED((ACC_ROWS,), jnp.float32))
    scratch.append(pltpu.SemaphoreType.DMA)

    return pl.kernel(
        functools.partial(_agg_kernel_body, with_count),
        mesh=mesh,
        out_type=tuple(out_type) if with_count else out_type[0],
        scratch_types=scratch,
    )


def _tc_layer_body(relu, a_ref, cnt_ref, x_ref, wl_ref, wr_ref, b_ref, out_ref):
    rec = 1.0 / jnp.maximum(cnt_ref[...], 1.0)
    mean = a_ref[...] * rec
    z = (jnp.dot(mean, wl_ref[...], preferred_element_type=jnp.float32)
         + jnp.dot(x_ref[...], wr_ref[...], preferred_element_type=jnp.float32)
         + b_ref[...])
    out_ref[...] = jnp.maximum(z, 0.0) if relu else z


@functools.lru_cache(maxsize=None)
def _make_tc_layer(relu):
    B = 1024
    grid = (N_PAD // B,)
    return pl.pallas_call(
        functools.partial(_tc_layer_body, relu),
        grid=grid,
        in_specs=[
            pl.BlockSpec((B, D), lambda i: (i, 0)),
            pl.BlockSpec((B, 1), lambda i: (i, 0)),
            pl.BlockSpec((B, D), lambda i: (i, 0)),
            pl.BlockSpec((D, D), lambda i: (0, 0)),
            pl.BlockSpec((D, D), lambda i: (0, 0)),
            pl.BlockSpec((1, D), lambda i: (0, 0)),
        ],
        out_specs=pl.BlockSpec((B, D), lambda i: (i, 0)),
        out_shape=jax.ShapeDtypeStruct((N_PAD, D), jnp.float32),
    )


def kernel(x, edge_index, W_l1, W_r1, b1, W_l2, W_r2, b2, W_l3, W_r3, b3):
    src = edge_index[0]
    dst = edge_index[1]
    pad_e = E_PAD - E_EDGES
    src_r = jnp.concatenate(
        [src, jnp.zeros((pad_e,), jnp.int32)]).reshape(NS, CHUNKS_PER_TILE, CHUNK)
    dst_r = jnp.concatenate(
        [dst, jnp.full((pad_e,), N_NODES, jnp.int32)]).reshape(NS, CHUNKS_PER_TILE, CHUNK)

    x_pad = jnp.pad(x, ((0, N_PAD - N_NODES), (0, 0)))

    a1, cnt = _make_agg(True)(x_pad, src_r, dst_r)
    cnt2 = cnt.reshape(N_PAD, 1)

    def pad_w(w):
        return jnp.pad(w, ((0, 0), (0, D - w.shape[1])))

    b1r = b1.reshape(1, D)
    b2r = b2.reshape(1, D)
    b3r = jnp.pad(b3, (0, D - b3.shape[0])).reshape(1, D)

    h1 = _make_tc_layer(True)(a1, cnt2, x_pad, W_l1, W_r1, b1r)
    a2 = _make_agg(False)(h1, src_r, dst_r)
    h2 = _make_tc_layer(True)(a2, cnt2, h1, W_l2, W_r2, b2r)
    a3 = _make_agg(False)(h2, src_r, dst_r)
    out = _make_tc_layer(False)(a3, cnt2, h2, pad_w(W_l3), pad_w(W_r3), b3r)
    return out[:N_NODES, :47]
